# baseline (speedup 1.0000x reference)
import functools

import jax
import jax.numpy as jnp
from jax import lax
from jax.experimental import pallas as pl
from jax.experimental.pallas import tpu as pltpu
from jax.experimental.pallas import tpu_sc as plsc

NC = 2
NS = 16
L = 16

LN2 = 0.6931471805599453
SQRT2 = 1.4142135623730951


def _log_1_to_k(s):
    bits = plsc.bitcast(s, jnp.int32)
    e = (bits >> 23) - 127
    mbits = (bits & jnp.int32(0x007FFFFF)) | jnp.int32(0x3F800000)
    m = plsc.bitcast(mbits, jnp.float32)
    big = m > jnp.float32(SQRT2)
    m = jnp.where(big, m * jnp.float32(0.5), m)
    e = jnp.where(big, e + 1, e)
    z = (m - jnp.float32(1.0)) / (m + jnp.float32(1.0))
    z2 = z * z
    p = jnp.float32(2.0) + z2 * (
        jnp.float32(2.0 / 3.0)
        + z2 * (jnp.float32(2.0 / 5.0) + z2 * (jnp.float32(2.0 / 7.0) + z2 * jnp.float32(2.0 / 9.0)))
    )
    return z * p + e.astype(jnp.float32) * jnp.float32(LN2)


def _make_sc_loss(bsz, m, k):
    rows = bsz * m
    nw = NC * NS
    rpw = rows // nw
    ch = 256
    nch = rpw // ch
    gpc = ch // L
    kv = k // L

    mesh = plsc.VectorSubcoreMesh(core_axis_name="c", subcore_axis_name="s")

    @functools.partial(
        pl.kernel,
        out_type=(
            jax.ShapeDtypeStruct((nw, L), jnp.float32),
            jax.ShapeDtypeStruct((nw, L), jnp.float32),
        ),
        mesh=mesh,
        compiler_params=pltpu.CompilerParams(needs_layout_passes=False),
        scratch_types=[
            pltpu.VMEM((2 * ch, k), jnp.float32),
            pltpu.VMEM((rpw,), jnp.int32),
            pltpu.VMEM((rpw + 8,), jnp.int32),
            pltpu.VMEM((rpw + 8,), jnp.int32),
            pltpu.VMEM((k,), jnp.float32),
            pltpu.VMEM((ch * 17,), jnp.float32),
            pltpu.VMEM((ch * 17,), jnp.float32),
            pltpu.VMEM((L,), jnp.float32),
            pltpu.VMEM((L,), jnp.float32),
            pltpu.SemaphoreType.DMA,
            pltpu.SemaphoreType.DMA,
        ],
    )
    def sc_loss(x_hbm, lab_hbm, msk_hbm, w_hbm, wnll_hbm, wsum_hbm,
                xbuf, idxbuf, labwin, mskwin, wbuf,
                st_m, st_s, st_wnll, st_wsum, sem_a, sem_b):
        wid = lax.axis_index("s") * NC + lax.axis_index("c")
        base = wid * rpw
        bb = base // m
        off = base + bb + 1
        al = pl.multiple_of((off // 8) * 8, 8)
        shift = off % 8
        pltpu.sync_copy(w_hbm, wbuf)
        pltpu.sync_copy(lab_hbm.at[pl.ds(al, rpw + 8)], labwin)
        pltpu.sync_copy(msk_hbm.at[pl.ds(al, rpw + 8)], mskwin)
        lanes = lax.iota(jnp.int32, L)
        zeros = jnp.zeros((L,), jnp.float32)

        @plsc.parallel_loop(0, rpw // L, 1, unroll=8)
        def _(t):
            idxbuf[pl.ds(t * L, L)] = jnp.full((L,), 0, jnp.int32) + base

        @plsc.parallel_loop(0, rpw // L, 1, unroll=4, carry=jnp.zeros((L,), jnp.int32))
        def offvec(t, off16):
            mk = mskwin[pl.ds(shift + t * L, L)] != 0
            pos = off16 + plsc.cumsum(mk.astype(jnp.int32)) - 1
            plsc.store_scatter(idxbuf, [pos], base + t * L + lanes, mask=mk)
            return off16 + plsc.all_reduce_population_count(mk)

        n = jnp.max(offvec)
        nchd = (n + (ch - 1)) // ch

        def dma_start(ci, slot, sem):
            half = ch // 2
            pltpu.make_async_copy(
                x_hbm.at[idxbuf.at[pl.ds(ci * ch, half)]],
                xbuf.at[pl.ds(slot * ch, half)], sem).start()
            pltpu.make_async_copy(
                x_hbm.at[idxbuf.at[pl.ds(ci * ch + half, half)]],
                xbuf.at[pl.ds(slot * ch + half, half)], sem).start()

        def dma_wait(slot, sem):
            pltpu.make_async_copy(
                x_hbm.at[pl.ds(0, ch)],
                xbuf.at[pl.ds(slot * ch, ch)], sem).wait()

        def compute_chunk(ci, slot, carry):
            soff = slot * ch

            @plsc.parallel_loop(0, ch, 1, unroll=4)
            def _(r):
                vs = [xbuf[soff + r, pl.ds(j * L, L)] for j in range(kv)]
                t = vs
                while len(t) > 1:
                    t = [jnp.maximum(t[2 * i], t[2 * i + 1]) for i in range(len(t) // 2)]
                bm = jnp.max(t[0]) + zeros
                es = [jnp.exp(v - bm) for v in vs]
                while len(es) > 1:
                    es = [es[2 * i] + es[2 * i + 1] for i in range(len(es) // 2)]
                bs = jnp.sum(es[0]) + zeros
                st_m[pl.ds(r * 17, L)] = bm
                st_s[pl.ds(r * 17, L)] = bs

            def group_body(g, carry2):
                a1, a2 = carry2
                slot16 = ci * ch + g * L + lanes
                rows16 = g * L + lanes
                idx16 = idxbuf[pl.ds(ci * ch + g * L, L)]
                gt16 = plsc.load_gather(labwin, [idx16 - (base - shift)])
                xg = plsc.load_gather(xbuf, [soff + rows16, gt16])
                wv = plsc.load_gather(wbuf, [gt16])
                mv = plsc.load_gather(st_m, [rows16 * 17])
                sv = plsc.load_gather(st_s, [rows16 * 17])
                w = jnp.where(slot16 < n, wv, jnp.float32(0.0))
                nll = _log_1_to_k(sv) + mv - xg
                return (a1 + w * nll, a2 + w)

            return lax.fori_loop(0, gpc, group_body, carry)

        def dma_start_d(ci, slot):
            @pl.when(slot == 0)
            def _():
                dma_start(ci, 0, sem_a)

            @pl.when(slot != 0)
            def _():
                dma_start(ci, 1, sem_b)

        def dma_wait_d(slot):
            @pl.when(slot == 0)
            def _():
                dma_wait(0, sem_a)

            @pl.when(slot != 0)
            def _():
                dma_wait(1, sem_b)

        @pl.when(0 < nchd)
        def _():
            dma_start(0, 0, sem_a)

        @pl.when(1 < nchd)
        def _():
            dma_start(1, 1, sem_b)

        def chunk_loop(ci, carry):
            slot = ci % 2
            dma_wait_d(slot)
            carry = compute_chunk(ci, slot, carry)

            @pl.when(ci + 2 < nchd)
            def _():
                dma_start_d(ci + 2, slot)

            return carry

        a_wnll, a_w = lax.fori_loop(0, nchd, chunk_loop, (zeros, zeros))
        st_wnll[...] = a_wnll
        st_wsum[...] = a_w
        pltpu.sync_copy(st_wnll, wnll_hbm.at[wid])
        pltpu.sync_copy(st_wsum, wsum_hbm.at[wid])

    return sc_loss


def kernel(outputs, lables, masks, weight):
    b, m, k = outputs.shape
    x = outputs.reshape(b * m, k)
    lab = lables.reshape(b * (m + 1))
    msk = masks.reshape(b * (m + 1))
    wnll, wsum = _make_sc_loss(b, m, k)(x, lab, msk, weight)
    return jnp.sum(wnll) / jnp.sum(wsum)


# speedup vs baseline: 1.0883x; 1.0883x over previous
import functools

import jax
import jax.numpy as jnp
from jax import lax
from jax.experimental import pallas as pl
from jax.experimental.pallas import tpu as pltpu
from jax.experimental.pallas import tpu_sc as plsc

NC = 2
NS = 16
L = 16

LN2 = 0.6931471805599453
SQRT2 = 1.4142135623730951


def _log_1_to_k(s):
    bits = plsc.bitcast(s, jnp.int32)
    e = (bits >> 23) - 127
    mbits = (bits & jnp.int32(0x007FFFFF)) | jnp.int32(0x3F800000)
    m = plsc.bitcast(mbits, jnp.float32)
    big = m > jnp.float32(SQRT2)
    m = jnp.where(big, m * jnp.float32(0.5), m)
    e = jnp.where(big, e + 1, e)
    z = (m - jnp.float32(1.0)) / (m + jnp.float32(1.0))
    z2 = z * z
    p = jnp.float32(2.0) + z2 * (
        jnp.float32(2.0 / 3.0)
        + z2 * (jnp.float32(2.0 / 5.0) + z2 * (jnp.float32(2.0 / 7.0) + z2 * jnp.float32(2.0 / 9.0)))
    )
    return z * p + e.astype(jnp.float32) * jnp.float32(LN2)


def _make_sc_loss(bsz, m, k):
    rows = bsz * m
    nw = NC * NS
    rpw = rows // nw
    ch = 256
    nch = rpw // ch
    gpc = ch // L
    kv = k // L

    mesh = plsc.VectorSubcoreMesh(core_axis_name="c", subcore_axis_name="s")

    @functools.partial(
        pl.kernel,
        out_type=(
            jax.ShapeDtypeStruct((nw, L), jnp.float32),
            jax.ShapeDtypeStruct((nw, L), jnp.float32),
        ),
        mesh=mesh,
        compiler_params=pltpu.CompilerParams(needs_layout_passes=False),
        scratch_types=[
            pltpu.VMEM((2 * ch * k,), jnp.float32),
            pltpu.VMEM((2 * (ch + 8),), jnp.int32),
            pltpu.VMEM((2 * (ch + 8),), jnp.int32),
            pltpu.VMEM((k,), jnp.float32),
            pltpu.VMEM((ch * 17,), jnp.float32),
            pltpu.VMEM((ch * 17,), jnp.float32),
            pltpu.VMEM((L,), jnp.float32),
            pltpu.VMEM((L,), jnp.float32),
            pltpu.SemaphoreType.DMA,
            pltpu.SemaphoreType.DMA,
        ],
    )
    def sc_loss(x_hbm, lab_hbm, msk_hbm, w_hbm, wnll_hbm, wsum_hbm,
                xbuf, gtbuf, kpbuf, wbuf,
                st_m, st_s, st_wnll, st_wsum, sem_a, sem_b):
        wid = lax.axis_index("s") * NC + lax.axis_index("c")
        base = wid * rpw
        bb = base // m
        pltpu.sync_copy(w_hbm, wbuf)
        lanes = lax.iota(jnp.int32, L)
        zeros = jnp.zeros((L,), jnp.float32)
        sems = (sem_a, sem_b)

        def dma_start(ci, slot, sem):
            st = base + ci * ch
            off = st + bb + 1
            al = pl.multiple_of((off // 8) * 8, 8)
            pltpu.make_async_copy(
                x_hbm.at[pl.ds(st * k, ch * k)],
                xbuf.at[pl.ds(slot * (ch * k), ch * k)], sem).start()
            pltpu.make_async_copy(
                lab_hbm.at[pl.ds(al, ch + 8)],
                gtbuf.at[pl.ds(slot * (ch + 8), ch + 8)], sem).start()
            pltpu.make_async_copy(
                msk_hbm.at[pl.ds(al, ch + 8)],
                kpbuf.at[pl.ds(slot * (ch + 8), ch + 8)], sem).start()

        def dma_wait(slot, sem):
            pltpu.make_async_copy(
                x_hbm.at[pl.ds(0, ch * k)],
                xbuf.at[pl.ds(slot * (ch * k), ch * k)], sem).wait()
            pltpu.make_async_copy(
                lab_hbm.at[pl.ds(0, ch + 8)],
                gtbuf.at[pl.ds(slot * (ch + 8), ch + 8)], sem).wait()
            pltpu.make_async_copy(
                msk_hbm.at[pl.ds(0, ch + 8)],
                kpbuf.at[pl.ds(slot * (ch + 8), ch + 8)], sem).wait()

        def compute_chunk(ci, slot, carry):
            shift = (base + ci * ch + bb + 1) % 8
            xoff = slot * (ch * k)
            goff = slot * (ch + 8)
            @plsc.parallel_loop(0, ch, 1, unroll=8)
            def _(r):
                rbase = xoff + r * k
                vs = [xbuf[pl.ds(rbase + j * L, L)] for j in range(kv)]
                t = vs
                while len(t) > 1:
                    t = [jnp.maximum(t[2 * i], t[2 * i + 1]) for i in range(len(t) // 2)]
                bm = jnp.max(t[0]) + zeros
                es = [jnp.exp(v - bm) for v in vs]
                while len(es) > 1:
                    es = [es[2 * i] + es[2 * i + 1] for i in range(len(es) // 2)]
                bs = jnp.sum(es[0]) + zeros
                st_m[pl.ds(r * 17, L)] = bm
                st_s[pl.ds(r * 17, L)] = bs

            def group_body(g, carry2):
                a1, a2 = carry2
                rows16 = g * L + lanes
                gt16 = gtbuf[pl.ds(goff + shift + g * L, L)]
                xg = plsc.load_gather(xbuf, [xoff + rows16 * k + gt16])
                wv = plsc.load_gather(wbuf, [gt16])
                mv = plsc.load_gather(st_m, [rows16 * 17])
                sv = plsc.load_gather(st_s, [rows16 * 17])
                kp = kpbuf[pl.ds(goff + shift + g * L, L)]
                w = jnp.where(kp != 0, wv, jnp.float32(0.0))
                nll = _log_1_to_k(sv) + mv - xg
                return (a1 + w * nll, a2 + w)

            return lax.fori_loop(0, gpc, group_body, carry)

        def dma_start_d(ci, slot):
            @pl.when(slot == 0)
            def _():
                dma_start(ci, 0, sem_a)

            @pl.when(slot != 0)
            def _():
                dma_start(ci, 1, sem_b)

        def dma_wait_d(slot):
            @pl.when(slot == 0)
            def _():
                dma_wait(0, sem_a)

            @pl.when(slot != 0)
            def _():
                dma_wait(1, sem_b)

        dma_start(0, 0, sem_a)
        dma_start(1, 1, sem_b)

        def chunk_loop(ci, carry):
            slot = ci % 2
            dma_wait_d(slot)
            carry = compute_chunk(ci, slot, carry)

            @pl.when(ci + 2 < nch)
            def _():
                dma_start_d(ci + 2, slot)

            return carry

        a_wnll, a_w = lax.fori_loop(0, nch, chunk_loop, (zeros, zeros))
        st_wnll[...] = a_wnll
        st_wsum[...] = a_w
        pltpu.sync_copy(st_wnll, wnll_hbm.at[wid])
        pltpu.sync_copy(st_wsum, wsum_hbm.at[wid])

    return sc_loss


def kernel(outputs, lables, masks, weight):
    b, m, k = outputs.shape
    x = outputs.reshape(b * m * k)
    lab = lables.reshape(b * (m + 1))
    msk = masks.reshape(b * (m + 1))
    wnll, wsum = _make_sc_loss(b, m, k)(x, lab, msk, weight)
    return jnp.sum(wnll) / jnp.sum(wsum)


# speedup vs baseline: 1.2445x; 1.1435x over previous
import functools

import jax
import jax.numpy as jnp
from jax import lax
from jax.experimental import pallas as pl
from jax.experimental.pallas import tpu as pltpu
from jax.experimental.pallas import tpu_sc as plsc

NC = 2
NS = 16
L = 16

LN2 = 0.6931471805599453
SQRT2 = 1.4142135623730951


def _log_1_to_k(s):
    bits = plsc.bitcast(s, jnp.int32)
    e = (bits >> 23) - 127
    mbits = (bits & jnp.int32(0x007FFFFF)) | jnp.int32(0x3F800000)
    m = plsc.bitcast(mbits, jnp.float32)
    big = m > jnp.float32(SQRT2)
    m = jnp.where(big, m * jnp.float32(0.5), m)
    e = jnp.where(big, e + 1, e)
    z = (m - jnp.float32(1.0)) / (m + jnp.float32(1.0))
    z2 = z * z
    p = jnp.float32(2.0) + z2 * (
        jnp.float32(2.0 / 3.0)
        + z2 * (jnp.float32(2.0 / 5.0) + z2 * (jnp.float32(2.0 / 7.0) + z2 * jnp.float32(2.0 / 9.0)))
    )
    return z * p + e.astype(jnp.float32) * jnp.float32(LN2)


def _make_sc_loss(bsz, m, k):
    rows = bsz * m
    nw = NC * NS
    rpw = rows // nw
    ch = 256
    nch = rpw // ch
    gpc = ch // L
    kv = k // L

    mesh = plsc.VectorSubcoreMesh(core_axis_name="c", subcore_axis_name="s")

    @functools.partial(
        pl.kernel,
        out_type=(
            jax.ShapeDtypeStruct((nw, L), jnp.float32),
            jax.ShapeDtypeStruct((nw, L), jnp.float32),
        ),
        mesh=mesh,
        compiler_params=pltpu.CompilerParams(needs_layout_passes=False),
        scratch_types=[
            pltpu.VMEM((2 * ch * k,), jnp.float32),
            pltpu.VMEM((rpw + 8,), jnp.int32),
            pltpu.VMEM((rpw + 8,), jnp.int32),
            pltpu.VMEM((k,), jnp.float32),
            pltpu.VMEM((ch * 17,), jnp.float32),
            pltpu.VMEM((ch * 17,), jnp.float32),
            pltpu.VMEM((L,), jnp.float32),
            pltpu.VMEM((L,), jnp.float32),
            pltpu.SemaphoreType.DMA,
            pltpu.SemaphoreType.DMA,
        ],
    )
    def sc_loss(x_hbm, lab_hbm, msk_hbm, w_hbm, wnll_hbm, wsum_hbm,
                xbuf, labwin, mskwin, wbuf,
                st_m, st_s, st_wnll, st_wsum, sem_a, sem_b):
        wid = lax.axis_index("s") * NC + lax.axis_index("c")
        base = wid * rpw
        bb = base // m
        off = base + bb + 1
        al = pl.multiple_of((off // 8) * 8, 8)
        shift = off % 8
        pltpu.sync_copy(w_hbm, wbuf)
        pltpu.sync_copy(lab_hbm.at[pl.ds(al, rpw + 8)], labwin)
        pltpu.sync_copy(msk_hbm.at[pl.ds(al, rpw + 8)], mskwin)
        lanes = lax.iota(jnp.int32, L)
        zeros = jnp.zeros((L,), jnp.float32)

        def dma_start(ci, slot, sem):
            st = base + ci * ch
            pltpu.make_async_copy(
                x_hbm.at[pl.ds(st * k, ch * k)],
                xbuf.at[pl.ds(slot * (ch * k), ch * k)], sem).start()

        def dma_wait(slot, sem):
            pltpu.make_async_copy(
                x_hbm.at[pl.ds(0, ch * k)],
                xbuf.at[pl.ds(slot * (ch * k), ch * k)], sem).wait()

        def compute_chunk(ci, slot, carry):
            xoff = slot * (ch * k)
            goff = ci * ch
            @plsc.parallel_loop(0, ch, 1, unroll=4)
            def _(r):
                rbase = xoff + r * k
                vs = [xbuf[pl.ds(rbase + j * L, L)] for j in range(kv)]
                t = vs
                while len(t) > 1:
                    t = [jnp.maximum(t[2 * i], t[2 * i + 1]) for i in range(len(t) // 2)]
                bm = jnp.max(t[0]) + zeros
                es = [jnp.exp(v - bm) for v in vs]
                while len(es) > 1:
                    es = [es[2 * i] + es[2 * i + 1] for i in range(len(es) // 2)]
                bs = jnp.sum(es[0]) + zeros
                st_m[pl.ds(r * 17, L)] = bm
                st_s[pl.ds(r * 17, L)] = bs

            def group_body(g, carry2):
                a1, a2 = carry2
                rows16 = g * L + lanes
                gt16 = labwin[pl.ds(goff + shift + g * L, L)]
                xg = plsc.load_gather(xbuf, [xoff + rows16 * k + gt16])
                wv = plsc.load_gather(wbuf, [gt16])
                mv = plsc.load_gather(st_m, [rows16 * 17])
                sv = plsc.load_gather(st_s, [rows16 * 17])
                kp = mskwin[pl.ds(goff + shift + g * L, L)]
                w = jnp.where(kp != 0, wv, jnp.float32(0.0))
                nll = _log_1_to_k(sv) + mv - xg
                return (a1 + w * nll, a2 + w)

            return lax.fori_loop(0, gpc, group_body, carry)

        def dma_start_d(ci, slot):
            @pl.when(slot == 0)
            def _():
                dma_start(ci, 0, sem_a)

            @pl.when(slot != 0)
            def _():
                dma_start(ci, 1, sem_b)

        def dma_wait_d(slot):
            @pl.when(slot == 0)
            def _():
                dma_wait(0, sem_a)

            @pl.when(slot != 0)
            def _():
                dma_wait(1, sem_b)

        dma_start(0, 0, sem_a)
        dma_start(1, 1, sem_b)

        def chunk_loop(ci, carry):
            slot = ci % 2
            dma_wait_d(slot)
            carry = compute_chunk(ci, slot, carry)

            @pl.when(ci + 2 < nch)
            def _():
                dma_start_d(ci + 2, slot)

            return carry

        a_wnll, a_w = lax.fori_loop(0, nch, chunk_loop, (zeros, zeros))
        st_wnll[...] = a_wnll
        st_wsum[...] = a_w
        pltpu.sync_copy(st_wnll, wnll_hbm.at[wid])
        pltpu.sync_copy(st_wsum, wsum_hbm.at[wid])

    return sc_loss


def kernel(outputs, lables, masks, weight):
    b, m, k = outputs.shape
    x = outputs.reshape(b * m * k)
    lab = lables.reshape(b * (m + 1))
    msk = masks.reshape(b * (m + 1))
    wnll, wsum = _make_sc_loss(b, m, k)(x, lab, msk, weight)
    return jnp.sum(wnll) / jnp.sum(wsum)


# speedup vs baseline: 1.3533x; 1.0874x over previous
import functools

import jax
import jax.numpy as jnp
from jax import lax
from jax.experimental import pallas as pl
from jax.experimental.pallas import tpu as pltpu
from jax.experimental.pallas import tpu_sc as plsc

NC = 2
NS = 16
L = 16

LN2 = 0.6931471805599453
SQRT2 = 1.4142135623730951


def _log_1_to_k(s):
    bits = plsc.bitcast(s, jnp.int32)
    e = (bits >> 23) - 127
    mbits = (bits & jnp.int32(0x007FFFFF)) | jnp.int32(0x3F800000)
    m = plsc.bitcast(mbits, jnp.float32)
    big = m > jnp.float32(SQRT2)
    m = jnp.where(big, m * jnp.float32(0.5), m)
    e = jnp.where(big, e + 1, e)
    z = (m - jnp.float32(1.0)) / (m + jnp.float32(1.0))
    z2 = z * z
    p = jnp.float32(2.0) + z2 * (
        jnp.float32(2.0 / 3.0)
        + z2 * (jnp.float32(2.0 / 5.0) + z2 * (jnp.float32(2.0 / 7.0) + z2 * jnp.float32(2.0 / 9.0)))
    )
    return z * p + e.astype(jnp.float32) * jnp.float32(LN2)


def _make_sc_loss(bsz, m, k):
    rows = bsz * m
    nw = NC * NS
    rpw = rows // nw
    ch = 256
    nch = rpw // ch
    gpc = ch // L
    kv = k // L

    mesh = plsc.VectorSubcoreMesh(core_axis_name="c", subcore_axis_name="s")

    @functools.partial(
        pl.kernel,
        out_type=jax.ShapeDtypeStruct((2, nw, L), jnp.float32),
        mesh=mesh,
        compiler_params=pltpu.CompilerParams(needs_layout_passes=False),
        scratch_types=[
            pltpu.VMEM((2 * ch * k,), jnp.float32),
            pltpu.VMEM((2 * (ch + 8),), jnp.int32),
            pltpu.VMEM((2 * (ch + 8),), jnp.int32),
            pltpu.VMEM((k,), jnp.float32),
            pltpu.VMEM((ch * 17,), jnp.float32),
            pltpu.VMEM((ch * 17,), jnp.float32),
            pltpu.VMEM((L,), jnp.float32),
            pltpu.VMEM((L,), jnp.float32),
            pltpu.SemaphoreType.DMA,
            pltpu.SemaphoreType.DMA,
        ],
    )
    def sc_loss(x_hbm, lab_hbm, msk_hbm, w_hbm, part_hbm,
                xbuf, gtbuf, kpbuf, wbuf,
                st_m, st_s, st_wnll, st_wsum, sem_a, sem_b):
        wid = lax.axis_index("s") * NC + lax.axis_index("c")
        base = wid * rpw
        bb = base // m
        lanes = lax.iota(jnp.int32, L)
        zeros = jnp.zeros((L,), jnp.float32)

        def dma_start(ci, slot, sem):
            st = base + ci * ch
            off = st + bb + 1
            al = pl.multiple_of((off // 8) * 8, 8)
            pltpu.make_async_copy(
                x_hbm.at[pl.ds(st * k, ch * k)],
                xbuf.at[pl.ds(slot * (ch * k), ch * k)], sem).start()
            pltpu.make_async_copy(
                lab_hbm.at[pl.ds(al, ch + 8)],
                gtbuf.at[pl.ds(slot * (ch + 8), ch + 8)], sem).start()
            pltpu.make_async_copy(
                msk_hbm.at[pl.ds(al, ch + 8)],
                kpbuf.at[pl.ds(slot * (ch + 8), ch + 8)], sem).start()

        def dma_wait(slot, sem):
            pltpu.make_async_copy(
                x_hbm.at[pl.ds(0, ch * k)],
                xbuf.at[pl.ds(slot * (ch * k), ch * k)], sem).wait()
            pltpu.make_async_copy(
                lab_hbm.at[pl.ds(0, ch + 8)],
                gtbuf.at[pl.ds(slot * (ch + 8), ch + 8)], sem).wait()
            pltpu.make_async_copy(
                msk_hbm.at[pl.ds(0, ch + 8)],
                kpbuf.at[pl.ds(slot * (ch + 8), ch + 8)], sem).wait()

        def compute_chunk(ci, slot, carry):
            shift = (base + ci * ch + bb + 1) % 8
            xoff = slot * (ch * k)
            goff = slot * (ch + 8)
            @plsc.parallel_loop(0, ch, 1, unroll=4)
            def _(r):
                rbase = xoff + r * k
                vs = [xbuf[pl.ds(rbase + j * L, L)] for j in range(kv)]
                t = vs
                while len(t) > 1:
                    t = [jnp.maximum(t[2 * i], t[2 * i + 1]) for i in range(len(t) // 2)]
                bm = jnp.max(t[0]) + zeros
                es = [jnp.exp(v - bm) for v in vs]
                while len(es) > 1:
                    es = [es[2 * i] + es[2 * i + 1] for i in range(len(es) // 2)]
                bs = jnp.sum(es[0]) + zeros
                st_m[pl.ds(r * 17, L)] = bm
                st_s[pl.ds(r * 17, L)] = bs

            def group_body(g, carry2):
                a1, a2 = carry2
                rows16 = g * L + lanes
                gt16 = gtbuf[pl.ds(goff + shift + g * L, L)]
                xg = plsc.load_gather(xbuf, [xoff + rows16 * k + gt16])
                wv = plsc.load_gather(wbuf, [gt16])
                mv = plsc.load_gather(st_m, [rows16 * 17])
                sv = plsc.load_gather(st_s, [rows16 * 17])
                kp = kpbuf[pl.ds(goff + shift + g * L, L)]
                w = jnp.where(kp != 0, wv, jnp.float32(0.0))
                nll = _log_1_to_k(sv) + mv - xg
                return (a1 + w * nll, a2 + w)

            return lax.fori_loop(0, gpc, group_body, carry)

        def dma_start_d(ci, slot):
            @pl.when(slot == 0)
            def _():
                dma_start(ci, 0, sem_a)

            @pl.when(slot != 0)
            def _():
                dma_start(ci, 1, sem_b)

        def dma_wait_d(slot):
            @pl.when(slot == 0)
            def _():
                dma_wait(0, sem_a)

            @pl.when(slot != 0)
            def _():
                dma_wait(1, sem_b)

        dma_start(0, 0, sem_a)
        dma_start(1, 1, sem_b)
        pltpu.sync_copy(w_hbm, wbuf)

        def chunk_loop(ci, carry):
            slot = ci % 2
            dma_wait_d(slot)
            carry = compute_chunk(ci, slot, carry)

            @pl.when(ci + 2 < nch)
            def _():
                dma_start_d(ci + 2, slot)

            return carry

        a_wnll, a_w = lax.fori_loop(0, nch, chunk_loop, (zeros, zeros))
        st_wnll[...] = a_wnll
        st_wsum[...] = a_w
        pltpu.sync_copy(st_wnll, part_hbm.at[0, wid])
        pltpu.sync_copy(st_wsum, part_hbm.at[1, wid])

    return sc_loss


def kernel(outputs, lables, masks, weight):
    b, m, k = outputs.shape
    x = outputs.reshape(b * m * k)
    lab = lables.reshape(b * (m + 1))
    msk = masks.reshape(b * (m + 1))
    part = _make_sc_loss(b, m, k)(x, lab, msk, weight)
    tot = jnp.sum(part.reshape(2, -1), axis=1)
    return tot[0] / tot[1]
